# baseline (speedup 1.0000x reference)
import functools

import jax
import jax.numpy as jnp
from jax import lax
from jax.experimental import pallas as pl
from jax.experimental.pallas import tpu as pltpu
from jax.experimental.pallas import tpu_sc as plsc

N_NODES = 10000
N_PAD = 10112
IN_CH = 768
OUT_CH = 64
N_EDGES = 160000
K = 128
NC = 2
NS = 16
NW = NC * NS
E_PAD = 163840
BPW = E_PAD // (K * NW)
RPT = N_PAD // NS
NBUF = 8
GRPS = BPW // NBUF

def _zero_rows(zeros_hbm, table_sh, base):
    nfull, rem = divmod(RPT, K)
    for t in range(nfull):
        pltpu.sync_copy(zeros_hbm, table_sh.at[pl.ds(base + t * K, K)])
    if rem:
        pltpu.sync_copy(zeros_hbm.at[pl.ds(0, rem)],
                        table_sh.at[pl.ds(base + nfull * K, rem)])


_mesh = plsc.VectorSubcoreMesh(core_axis_name="c", subcore_axis_name="s")
_sc_params = pltpu.CompilerParams(use_tc_tiling_on_sc=False)


@functools.partial(
    pl.kernel,
    out_type=jax.ShapeDtypeStruct((NC, N_PAD, 16), jnp.float32),
    mesh=_mesh,
    compiler_params=_sc_params,
    scratch_types=[
        pltpu.VMEM((BPW, K), jnp.int32),
        pltpu.VMEM((K, 16), jnp.float32),
        pltpu.VMEM_SHARED((N_PAD, 16), jnp.float32),
        pltpu.SemaphoreType.DMA,
    ],
)
def _sc_degree(dst_hbm, ones_hbm, zeros_hbm, out_hbm, dst_v, ones_v, deg_sh, sem):
    cid = lax.axis_index("c")
    sid = lax.axis_index("s")
    wid = sid * NC + cid
    _zero_rows(zeros_hbm, deg_sh, sid * RPT)
    pltpu.sync_copy(ones_hbm, ones_v)
    pltpu.sync_copy(dst_hbm.at[wid], dst_v)
    plsc.subcore_barrier()
    for j in range(BPW):
        pltpu.async_copy(ones_v, deg_sh.at[dst_v.at[j]], sem, add=True)
    for j in range(BPW):
        pltpu.make_async_copy(ones_v, deg_sh.at[pl.ds(0, K)], sem).wait()
    plsc.subcore_barrier()
    pltpu.sync_copy(
        deg_sh.at[pl.ds(sid * RPT, RPT)],
        out_hbm.at[cid, pl.ds(sid * RPT, RPT)],
    )


@functools.partial(
    pl.kernel,
    out_type=jax.ShapeDtypeStruct((NC, N_PAD, OUT_CH), jnp.float32),
    mesh=_mesh,
    compiler_params=_sc_params,
    scratch_types=[
        pltpu.VMEM((BPW, K), jnp.int32),
        pltpu.VMEM((BPW, K), jnp.int32),
        pltpu.VMEM((NBUF, K, OUT_CH), jnp.float32),
        pltpu.VMEM_SHARED((N_PAD, OUT_CH), jnp.float32),
    ] + [pltpu.SemaphoreType.DMA] * (2 * NBUF),
)
def _sc_aggregate(src_hbm, dst_hbm, hs_hbm, zeros_hbm, out_hbm,
                  src_v, dst_v, rows_v, agg_sh, *sems):
    gsems = sems[:NBUF]
    ssems = sems[NBUF:]
    cid = lax.axis_index("c")
    sid = lax.axis_index("s")
    wid = sid * NC + cid
    _zero_rows(zeros_hbm, agg_sh, sid * RPT)
    pltpu.sync_copy(src_hbm.at[wid], src_v)
    pltpu.sync_copy(dst_hbm.at[wid], dst_v)

    def gather(j, b):
        pltpu.async_copy(hs_hbm.at[src_v.at[j]], rows_v.at[b], gsems[b])

    for b in range(NBUF):
        gather(b, b)
    plsc.subcore_barrier()

    def grp(g, carry):
        for b in range(NBUF):
            j = g * NBUF + b
            pltpu.make_async_copy(hs_hbm.at[pl.ds(0, K)], rows_v.at[b], gsems[b]).wait()
            pltpu.async_copy(rows_v.at[b], agg_sh.at[dst_v.at[j]], ssems[b], add=True)
        for b in range(NBUF):
            pltpu.make_async_copy(rows_v.at[b], agg_sh.at[pl.ds(0, K)], ssems[b]).wait()

            @pl.when(g + 1 < GRPS)
            def _():
                gather((g + 1) * NBUF + b, b)

        return carry

    lax.fori_loop(0, GRPS, grp, 0)
    plsc.subcore_barrier()
    pltpu.sync_copy(
        agg_sh.at[pl.ds(sid * RPT, RPT)],
        out_hbm.at[cid, pl.ds(sid * RPT, RPT)],
    )


_RB = 1000


def _deg_block(deg_ref):
    return deg_ref[0, :, 0:1] + deg_ref[1, :, 0:1] + 1.0


def _hs_body(x_ref, w_ref, deg_ref, hs_ref):
    dis = lax.rsqrt(_deg_block(deg_ref))
    h = jnp.dot(x_ref[...], w_ref[...], preferred_element_type=jnp.float32)
    hs_ref[...] = h * dis


def _tc_hs(x, w, deg_parts):
    grid = (N_NODES // _RB,)
    return pl.pallas_call(
        _hs_body,
        grid=grid,
        in_specs=[
            pl.BlockSpec((_RB, IN_CH), lambda i: (i, 0)),
            pl.BlockSpec((IN_CH, OUT_CH), lambda i: (0, 0)),
            pl.BlockSpec((2, _RB, 16), lambda i: (0, i, 0)),
        ],
        out_specs=pl.BlockSpec((_RB, OUT_CH), lambda i: (i, 0)),
        out_shape=jax.ShapeDtypeStruct((N_NODES, OUT_CH), jnp.float32),
    )(x, w, deg_parts)


def _epi_body(agg_ref, hs_ref, deg_ref, b_ref, out_ref):
    dis = lax.rsqrt(_deg_block(deg_ref))
    s = (agg_ref[0] + agg_ref[1] + hs_ref[...]) * dis + b_ref[...]
    s = jnp.maximum(s, 0.0)
    m = jnp.max(s, axis=-1, keepdims=True)
    lse = jnp.log(jnp.sum(jnp.exp(s - m), axis=-1, keepdims=True)) + m
    out_ref[...] = s - lse


def _tc_epilogue(agg_parts, hs, deg_parts, b):
    grid = (N_NODES // _RB,)
    return pl.pallas_call(
        _epi_body,
        grid=grid,
        in_specs=[
            pl.BlockSpec((2, _RB, OUT_CH), lambda i: (0, i, 0)),
            pl.BlockSpec((_RB, OUT_CH), lambda i: (i, 0)),
            pl.BlockSpec((2, _RB, 16), lambda i: (0, i, 0)),
            pl.BlockSpec((1, OUT_CH), lambda i: (0, 0)),
        ],
        out_specs=pl.BlockSpec((_RB, OUT_CH), lambda i: (i, 0)),
        out_shape=jax.ShapeDtypeStruct((N_NODES, OUT_CH), jnp.float32),
    )(agg_parts, hs, deg_parts, b)


def kernel(x, edge_index, W, b):
    ei = edge_index.astype(jnp.int32)
    pad = E_PAD - N_EDGES
    src = jnp.concatenate([ei[0], jnp.zeros((pad,), jnp.int32)])
    dst = jnp.concatenate([ei[1], jnp.full((pad,), N_NODES, jnp.int32)])
    src = src.reshape(NW, BPW, K)
    dst = dst.reshape(NW, BPW, K)
    src = src + (jnp.arange(NW, dtype=jnp.int32) % NC)[:, None, None] * N_NODES

    ones_rows = jnp.ones((K, 16), jnp.float32)
    zeros16 = jnp.zeros((K, 16), jnp.float32)
    zeros64 = jnp.zeros((K, OUT_CH), jnp.float32)

    deg_parts = _sc_degree(dst, ones_rows, zeros16)
    hs = _tc_hs(x, W, deg_parts)
    hs2 = jnp.concatenate([hs, hs], axis=0)
    agg_parts = _sc_aggregate(src, dst, hs2, zeros64)
    return _tc_epilogue(agg_parts, hs, deg_parts, b.reshape(1, OUT_CH))


# speedup vs baseline: 1.5468x; 1.5468x over previous
import functools

import jax
import jax.numpy as jnp
from jax import lax
from jax.experimental import pallas as pl
from jax.experimental.pallas import tpu as pltpu
from jax.experimental.pallas import tpu_sc as plsc

N_NODES = 10000
N_PAD = 10112
IN_CH = 768
OUT_CH = 64
N_EDGES = 160000
K = 128
NC = 2
NS = 16
NW = NC * NS
E_PAD = 163840
BPW = E_PAD // (K * NW)
RPT = N_PAD // NS
NBUF = 8
GRPS = BPW // NBUF

def _zero_rows(zeros_hbm, table_sh, base):
    nfull, rem = divmod(RPT, K)
    for t in range(nfull):
        pltpu.sync_copy(zeros_hbm, table_sh.at[pl.ds(base + t * K, K)])
    if rem:
        pltpu.sync_copy(zeros_hbm.at[pl.ds(0, rem)],
                        table_sh.at[pl.ds(base + nfull * K, rem)])


_mesh = plsc.VectorSubcoreMesh(core_axis_name="c", subcore_axis_name="s")
_sc_params = pltpu.CompilerParams(use_tc_tiling_on_sc=False)


@functools.partial(
    pl.kernel,
    out_type=jax.ShapeDtypeStruct((NC, N_PAD, 16), jnp.float32),
    mesh=_mesh,
    compiler_params=_sc_params,
    scratch_types=[
        pltpu.VMEM((BPW, K), jnp.int32),
        pltpu.VMEM((K, 16), jnp.float32),
        pltpu.VMEM_SHARED((N_PAD, 16), jnp.float32),
        pltpu.SemaphoreType.DMA,
    ],
)
def _sc_degree(dst_hbm, ones_hbm, zeros_hbm, out_hbm, dst_v, ones_v, deg_sh, sem):
    cid = lax.axis_index("c")
    sid = lax.axis_index("s")
    wid = sid * NC + cid
    _zero_rows(zeros_hbm, deg_sh, sid * RPT)
    pltpu.sync_copy(ones_hbm, ones_v)
    pltpu.sync_copy(dst_hbm.at[wid], dst_v)
    plsc.subcore_barrier()
    for j in range(BPW):
        pltpu.async_copy(ones_v, deg_sh.at[dst_v.at[j]], sem, add=True)
    for j in range(BPW):
        pltpu.make_async_copy(ones_v, deg_sh.at[pl.ds(0, K)], sem).wait()
    plsc.subcore_barrier()
    pltpu.sync_copy(
        deg_sh.at[pl.ds(sid * RPT, RPT)],
        out_hbm.at[cid, pl.ds(sid * RPT, RPT)],
    )


HCH = OUT_CH // 2


@functools.partial(
    pl.kernel,
    out_type=jax.ShapeDtypeStruct((2, NC, N_PAD, HCH), jnp.float32),
    mesh=_mesh,
    compiler_params=_sc_params,
    scratch_types=[
        pltpu.VMEM((BPW, K), jnp.int32),
        pltpu.VMEM((BPW, K), jnp.int32),
        pltpu.VMEM((NBUF, K, HCH), jnp.float32),
        pltpu.VMEM_SHARED((N_PAD, HCH), jnp.float32),
        pltpu.VMEM_SHARED((N_PAD, HCH), jnp.float32),
    ] + [pltpu.SemaphoreType.DMA] * (2 * NBUF),
)
def _sc_aggregate(src_hbm, dst_hbm, hs0_hbm, hs1_hbm, zeros_hbm, out_hbm,
                  src_v, dst_v, rows_v, agg_sh, hs_sh, *sems):
    gsems = sems[:NBUF]
    ssems = sems[NBUF:]
    cid = lax.axis_index("c")
    sid = lax.axis_index("s")
    wid = sid * NC + cid
    pltpu.sync_copy(src_hbm.at[wid], src_v)
    pltpu.sync_copy(dst_hbm.at[wid], dst_v)

    nfull = N_NODES // RPT
    tail = N_NODES - nfull * RPT
    for phase, hs_hbm in enumerate((hs0_hbm, hs1_hbm)):
        _zero_rows(zeros_hbm, agg_sh, sid * RPT)

        @pl.when(sid < nfull)
        def _():
            pltpu.sync_copy(hs_hbm.at[pl.ds(sid * RPT, RPT)],
                            hs_sh.at[pl.ds(sid * RPT, RPT)])

        @pl.when(sid == nfull)
        def _():
            pltpu.sync_copy(hs_hbm.at[pl.ds(nfull * RPT, tail)],
                            hs_sh.at[pl.ds(nfull * RPT, tail)])

        plsc.subcore_barrier()

        def gather(j, b):
            pltpu.async_copy(hs_sh.at[src_v.at[j]], rows_v.at[b], gsems[b])

        for b in range(NBUF):
            gather(b, b)

        def grp(g, carry):
            for b in range(NBUF):
                j = g * NBUF + b
                pltpu.make_async_copy(hs0_hbm.at[pl.ds(0, K)], rows_v.at[b],
                                      gsems[b]).wait()
                pltpu.async_copy(rows_v.at[b], agg_sh.at[dst_v.at[j]],
                                 ssems[b], add=True)
            for b in range(NBUF):
                pltpu.make_async_copy(rows_v.at[b], agg_sh.at[pl.ds(0, K)],
                                      ssems[b]).wait()

                @pl.when(g + 1 < GRPS)
                def _():
                    gather((g + 1) * NBUF + b, b)

            return carry

        lax.fori_loop(0, GRPS, grp, 0)
        plsc.subcore_barrier()
        pltpu.sync_copy(
            agg_sh.at[pl.ds(sid * RPT, RPT)],
            out_hbm.at[phase, cid, pl.ds(sid * RPT, RPT)],
        )


_RB = 1000


def _deg_block(deg_ref):
    return deg_ref[0, :, 0:1] + deg_ref[1, :, 0:1] + 1.0


def _hs_body(x_ref, w_ref, deg_ref, hs0_ref, hs1_ref):
    dis = lax.rsqrt(_deg_block(deg_ref))
    h = jnp.dot(x_ref[...], w_ref[...], preferred_element_type=jnp.float32)
    hs = h * dis
    hs0_ref[...] = hs[:, :HCH]
    hs1_ref[...] = hs[:, HCH:]


def _tc_hs(x, w, deg_parts):
    grid = (N_NODES // _RB,)
    half = jax.ShapeDtypeStruct((N_NODES, HCH), jnp.float32)
    return pl.pallas_call(
        _hs_body,
        grid=grid,
        in_specs=[
            pl.BlockSpec((_RB, IN_CH), lambda i: (i, 0)),
            pl.BlockSpec((IN_CH, OUT_CH), lambda i: (0, 0)),
            pl.BlockSpec((2, _RB, 16), lambda i: (0, i, 0)),
        ],
        out_specs=[
            pl.BlockSpec((_RB, HCH), lambda i: (i, 0)),
            pl.BlockSpec((_RB, HCH), lambda i: (i, 0)),
        ],
        out_shape=[half, half],
    )(x, w, deg_parts)


def _epi_body(agg_ref, hs0_ref, hs1_ref, deg_ref, b_ref, out_ref):
    dis = lax.rsqrt(_deg_block(deg_ref))
    agg = jnp.concatenate(
        [agg_ref[0, 0] + agg_ref[0, 1], agg_ref[1, 0] + agg_ref[1, 1]], axis=-1)
    hs = jnp.concatenate([hs0_ref[...], hs1_ref[...]], axis=-1)
    s = (agg + hs) * dis + b_ref[...]
    s = jnp.maximum(s, 0.0)
    m = jnp.max(s, axis=-1, keepdims=True)
    lse = jnp.log(jnp.sum(jnp.exp(s - m), axis=-1, keepdims=True)) + m
    out_ref[...] = s - lse


def _tc_epilogue(agg_parts, hs0, hs1, deg_parts, b):
    grid = (N_NODES // _RB,)
    return pl.pallas_call(
        _epi_body,
        grid=grid,
        in_specs=[
            pl.BlockSpec((2, 2, _RB, HCH), lambda i: (0, 0, i, 0)),
            pl.BlockSpec((_RB, HCH), lambda i: (i, 0)),
            pl.BlockSpec((_RB, HCH), lambda i: (i, 0)),
            pl.BlockSpec((2, _RB, 16), lambda i: (0, i, 0)),
            pl.BlockSpec((1, OUT_CH), lambda i: (0, 0)),
        ],
        out_specs=pl.BlockSpec((_RB, OUT_CH), lambda i: (i, 0)),
        out_shape=jax.ShapeDtypeStruct((N_NODES, OUT_CH), jnp.float32),
    )(agg_parts, hs0, hs1, deg_parts, b)


def kernel(x, edge_index, W, b):
    ei = edge_index.astype(jnp.int32)
    pad = E_PAD - N_EDGES
    src = jnp.concatenate([ei[0], jnp.zeros((pad,), jnp.int32)])
    dst = jnp.concatenate([ei[1], jnp.full((pad,), N_NODES, jnp.int32)])
    src = src.reshape(NW, BPW, K)
    dst = dst.reshape(NW, BPW, K)

    ones_rows = jnp.ones((K, 16), jnp.float32)
    zeros16 = jnp.zeros((K, 16), jnp.float32)
    zerosh = jnp.zeros((K, HCH), jnp.float32)

    deg_parts = _sc_degree(dst, ones_rows, zeros16)
    hs0, hs1 = _tc_hs(x, W, deg_parts)
    agg_parts = _sc_aggregate(src, dst, hs0, hs1, zerosh)
    return _tc_epilogue(agg_parts, hs0, hs1, deg_parts, b.reshape(1, OUT_CH))


# speedup vs baseline: 1.5595x; 1.0082x over previous
import functools

import jax
import jax.numpy as jnp
from jax import lax
from jax.experimental import pallas as pl
from jax.experimental.pallas import tpu as pltpu
from jax.experimental.pallas import tpu_sc as plsc

N_NODES = 10000
N_PAD = 10112
IN_CH = 768
OUT_CH = 64
N_EDGES = 160000
K = 128
NC = 2
NS = 16
NW = NC * NS
E_PAD = 163840
BPW = E_PAD // (K * NW)
RPT = N_PAD // NS
NBUF = 8
GRPS = BPW // NBUF

def _zero_rows(zeros_hbm, table_sh, base):
    nfull, rem = divmod(RPT, K)
    for t in range(nfull):
        pltpu.sync_copy(zeros_hbm, table_sh.at[pl.ds(base + t * K, K)])
    if rem:
        pltpu.sync_copy(zeros_hbm.at[pl.ds(0, rem)],
                        table_sh.at[pl.ds(base + nfull * K, rem)])


_mesh = plsc.VectorSubcoreMesh(core_axis_name="c", subcore_axis_name="s")
_sc_params = pltpu.CompilerParams(use_tc_tiling_on_sc=False)


@functools.partial(
    pl.kernel,
    out_type=jax.ShapeDtypeStruct((NC, N_PAD, 16), jnp.float32),
    mesh=_mesh,
    compiler_params=_sc_params,
    scratch_types=[
        pltpu.VMEM((BPW, K), jnp.int32),
        pltpu.VMEM((K, 16), jnp.float32),
        pltpu.VMEM_SHARED((N_PAD, 16), jnp.float32),
        pltpu.SemaphoreType.DMA,
    ],
)
def _sc_degree(dst_hbm, ones_hbm, zeros_hbm, out_hbm, dst_v, ones_v, deg_sh, sem):
    cid = lax.axis_index("c")
    sid = lax.axis_index("s")
    wid = sid * NC + cid
    _zero_rows(zeros_hbm, deg_sh, sid * RPT)
    pltpu.sync_copy(ones_hbm, ones_v)
    pltpu.sync_copy(dst_hbm.at[wid], dst_v)
    plsc.subcore_barrier()
    for j in range(BPW):
        pltpu.async_copy(ones_v, deg_sh.at[dst_v.at[j]], sem, add=True)
    for j in range(BPW):
        pltpu.make_async_copy(ones_v, deg_sh.at[pl.ds(0, K)], sem).wait()
    plsc.subcore_barrier()
    pltpu.sync_copy(
        deg_sh.at[pl.ds(sid * RPT, RPT)],
        out_hbm.at[cid, pl.ds(sid * RPT, RPT)],
    )


HCH = OUT_CH // 2


@functools.partial(
    pl.kernel,
    out_type=jax.ShapeDtypeStruct((2, NC, N_PAD, HCH), jnp.float32),
    mesh=_mesh,
    compiler_params=_sc_params,
    scratch_types=[
        pltpu.VMEM((BPW, K), jnp.int32),
        pltpu.VMEM((BPW, K), jnp.int32),
        pltpu.VMEM((NBUF, K, HCH), jnp.float32),
        pltpu.VMEM_SHARED((N_PAD, HCH), jnp.float32),
        pltpu.VMEM_SHARED((N_PAD, HCH), jnp.float32),
    ] + [pltpu.SemaphoreType.DMA] * (2 * NBUF),
)
def _sc_aggregate(src_hbm, dst_hbm, hs0_hbm, hs1_hbm, zeros_hbm, out_hbm,
                  src_v, dst_v, rows_v, agg_sh, hs_sh, *sems):
    gsems = sems[:NBUF]
    ssems = sems[NBUF:]
    cid = lax.axis_index("c")
    sid = lax.axis_index("s")
    wid = sid * NC + cid
    pltpu.sync_copy(src_hbm.at[wid], src_v)
    pltpu.sync_copy(dst_hbm.at[wid], dst_v)

    nfull = N_NODES // RPT
    tail = N_NODES - nfull * RPT
    for phase, hs_hbm in enumerate((hs0_hbm, hs1_hbm)):
        _zero_rows(zeros_hbm, agg_sh, sid * RPT)

        @pl.when(sid < nfull)
        def _():
            pltpu.sync_copy(hs_hbm.at[pl.ds(sid * RPT, RPT)],
                            hs_sh.at[pl.ds(sid * RPT, RPT)])

        @pl.when(sid == nfull)
        def _():
            pltpu.sync_copy(hs_hbm.at[pl.ds(nfull * RPT, tail)],
                            hs_sh.at[pl.ds(nfull * RPT, tail)])

        plsc.subcore_barrier()

        def gather(j, b):
            pltpu.async_copy(hs_sh.at[src_v.at[j]], rows_v.at[b], gsems[b])

        for b in range(NBUF):
            gather(b, b)

        def grp(g, carry):
            for b in range(NBUF):
                j = g * NBUF + b
                pltpu.make_async_copy(hs0_hbm.at[pl.ds(0, K)], rows_v.at[b],
                                      gsems[b]).wait()
                pltpu.async_copy(rows_v.at[b], agg_sh.at[dst_v.at[j]],
                                 ssems[b], add=True)
            for b in range(NBUF):
                pltpu.make_async_copy(rows_v.at[b], agg_sh.at[pl.ds(0, K)],
                                      ssems[b]).wait()

                @pl.when(g + 1 < GRPS)
                def _():
                    gather((g + 1) * NBUF + b, b)

            return carry

        lax.fori_loop(0, GRPS, grp, 0)
        plsc.subcore_barrier()
        pltpu.sync_copy(
            agg_sh.at[pl.ds(sid * RPT, RPT)],
            out_hbm.at[phase, cid, pl.ds(sid * RPT, RPT)],
        )


_RB = 1000


def _deg_block(deg_ref):
    return deg_ref[0, :, 0:1] + deg_ref[1, :, 0:1] + 1.0


def _mm_body(x_ref, w_ref, h_ref):
    h_ref[...] = jnp.dot(x_ref[...], w_ref[...],
                         preferred_element_type=jnp.float32)


def _tc_matmul(x, w):
    grid = (N_NODES // _RB,)
    return pl.pallas_call(
        _mm_body,
        grid=grid,
        in_specs=[
            pl.BlockSpec((_RB, IN_CH), lambda i: (i, 0)),
            pl.BlockSpec((IN_CH, OUT_CH), lambda i: (0, 0)),
        ],
        out_specs=pl.BlockSpec((_RB, OUT_CH), lambda i: (i, 0)),
        out_shape=jax.ShapeDtypeStruct((N_NODES, OUT_CH), jnp.float32),
    )(x, w)


def _hs_body(h_ref, deg_ref, hs0_ref, hs1_ref):
    dis = lax.rsqrt(_deg_block(deg_ref))
    hs = h_ref[...] * dis
    hs0_ref[...] = hs[:, :HCH]
    hs1_ref[...] = hs[:, HCH:]


def _tc_hs(h, deg_parts):
    grid = (N_NODES // _RB,)
    half = jax.ShapeDtypeStruct((N_NODES, HCH), jnp.float32)
    return pl.pallas_call(
        _hs_body,
        grid=grid,
        in_specs=[
            pl.BlockSpec((_RB, OUT_CH), lambda i: (i, 0)),
            pl.BlockSpec((2, _RB, 16), lambda i: (0, i, 0)),
        ],
        out_specs=[
            pl.BlockSpec((_RB, HCH), lambda i: (i, 0)),
            pl.BlockSpec((_RB, HCH), lambda i: (i, 0)),
        ],
        out_shape=[half, half],
    )(h, deg_parts)


def _epi_body(agg_ref, hs0_ref, hs1_ref, deg_ref, b_ref, out_ref):
    dis = lax.rsqrt(_deg_block(deg_ref))
    agg = jnp.concatenate(
        [agg_ref[0, 0] + agg_ref[0, 1], agg_ref[1, 0] + agg_ref[1, 1]], axis=-1)
    hs = jnp.concatenate([hs0_ref[...], hs1_ref[...]], axis=-1)
    s = (agg + hs) * dis + b_ref[...]
    s = jnp.maximum(s, 0.0)
    m = jnp.max(s, axis=-1, keepdims=True)
    lse = jnp.log(jnp.sum(jnp.exp(s - m), axis=-1, keepdims=True)) + m
    out_ref[...] = s - lse


def _tc_epilogue(agg_parts, hs0, hs1, deg_parts, b):
    grid = (N_NODES // _RB,)
    return pl.pallas_call(
        _epi_body,
        grid=grid,
        in_specs=[
            pl.BlockSpec((2, 2, _RB, HCH), lambda i: (0, 0, i, 0)),
            pl.BlockSpec((_RB, HCH), lambda i: (i, 0)),
            pl.BlockSpec((_RB, HCH), lambda i: (i, 0)),
            pl.BlockSpec((2, _RB, 16), lambda i: (0, i, 0)),
            pl.BlockSpec((1, OUT_CH), lambda i: (0, 0)),
        ],
        out_specs=pl.BlockSpec((_RB, OUT_CH), lambda i: (i, 0)),
        out_shape=jax.ShapeDtypeStruct((N_NODES, OUT_CH), jnp.float32),
    )(agg_parts, hs0, hs1, deg_parts, b)


def kernel(x, edge_index, W, b):
    ei = edge_index.astype(jnp.int32)
    pad = E_PAD - N_EDGES
    src = jnp.concatenate([ei[0], jnp.zeros((pad,), jnp.int32)])
    dst = jnp.concatenate([ei[1], jnp.full((pad,), N_NODES, jnp.int32)])
    src = src.reshape(NW, BPW, K)
    dst = dst.reshape(NW, BPW, K)

    ones_rows = jnp.ones((K, 16), jnp.float32)
    zeros16 = jnp.zeros((K, 16), jnp.float32)
    zerosh = jnp.zeros((K, HCH), jnp.float32)

    h = _tc_matmul(x, W)
    deg_parts = _sc_degree(dst, ones_rows, zeros16)
    hs0, hs1 = _tc_hs(h, deg_parts)
    agg_parts = _sc_aggregate(src, dst, hs0, hs1, zerosh)
    return _tc_epilogue(agg_parts, hs0, hs1, deg_parts, b.reshape(1, OUT_CH))


# speedup vs baseline: 1.5736x; 1.0091x over previous
import functools

import jax
import jax.numpy as jnp
from jax import lax
from jax.experimental import pallas as pl
from jax.experimental.pallas import tpu as pltpu
from jax.experimental.pallas import tpu_sc as plsc

N_NODES = 10000
N_PAD = 10112
IN_CH = 768
OUT_CH = 64
N_EDGES = 160000
K = 128
NC = 2
NS = 16
NW = NC * NS
E_PAD = 163840
BPW = E_PAD // (K * NW)
RPT = N_PAD // NS
NBUF = 10
GRPS = BPW // NBUF

def _zero_rows(zeros_hbm, table_sh, base):
    nfull, rem = divmod(RPT, K)
    for t in range(nfull):
        pltpu.sync_copy(zeros_hbm, table_sh.at[pl.ds(base + t * K, K)])
    if rem:
        pltpu.sync_copy(zeros_hbm.at[pl.ds(0, rem)],
                        table_sh.at[pl.ds(base + nfull * K, rem)])


_mesh = plsc.VectorSubcoreMesh(core_axis_name="c", subcore_axis_name="s")
_sc_params = pltpu.CompilerParams(use_tc_tiling_on_sc=False)


@functools.partial(
    pl.kernel,
    out_type=jax.ShapeDtypeStruct((NC, N_PAD, 16), jnp.float32),
    mesh=_mesh,
    compiler_params=_sc_params,
    scratch_types=[
        pltpu.VMEM((BPW, K), jnp.int32),
        pltpu.VMEM((K, 16), jnp.float32),
        pltpu.VMEM_SHARED((N_PAD, 16), jnp.float32),
        pltpu.SemaphoreType.DMA,
    ],
)
def _sc_degree(dst_hbm, ones_hbm, zeros_hbm, out_hbm, dst_v, ones_v, deg_sh, sem):
    cid = lax.axis_index("c")
    sid = lax.axis_index("s")
    wid = sid * NC + cid
    _zero_rows(zeros_hbm, deg_sh, sid * RPT)
    pltpu.sync_copy(ones_hbm, ones_v)
    pltpu.sync_copy(dst_hbm.at[wid], dst_v)
    plsc.subcore_barrier()
    for j in range(BPW):
        pltpu.async_copy(ones_v, deg_sh.at[dst_v.at[j]], sem, add=True)
    for j in range(BPW):
        pltpu.make_async_copy(ones_v, deg_sh.at[pl.ds(0, K)], sem).wait()
    plsc.subcore_barrier()
    pltpu.sync_copy(
        deg_sh.at[pl.ds(sid * RPT, RPT)],
        out_hbm.at[cid, pl.ds(sid * RPT, RPT)],
    )


HCH = OUT_CH // 2


@functools.partial(
    pl.kernel,
    out_type=jax.ShapeDtypeStruct((2, NC, N_PAD, HCH), jnp.float32),
    mesh=_mesh,
    compiler_params=_sc_params,
    scratch_types=[
        pltpu.VMEM((BPW, K), jnp.int32),
        pltpu.VMEM((BPW, K), jnp.int32),
        pltpu.VMEM((NBUF, K, HCH), jnp.float32),
        pltpu.VMEM_SHARED((N_PAD, HCH), jnp.float32),
        pltpu.VMEM_SHARED((N_PAD, HCH), jnp.float32),
    ] + [pltpu.SemaphoreType.DMA] * (2 * NBUF),
)
def _sc_aggregate(src_hbm, dst_hbm, hs0_hbm, hs1_hbm, zeros_hbm, out_hbm,
                  src_v, dst_v, rows_v, agg_sh, hs_sh, *sems):
    gsems = sems[:NBUF]
    ssems = sems[NBUF:]
    cid = lax.axis_index("c")
    sid = lax.axis_index("s")
    wid = sid * NC + cid
    pltpu.sync_copy(src_hbm.at[wid], src_v)
    pltpu.sync_copy(dst_hbm.at[wid], dst_v)

    nfull = N_NODES // RPT
    tail = N_NODES - nfull * RPT
    for phase, hs_hbm in enumerate((hs0_hbm, hs1_hbm)):
        _zero_rows(zeros_hbm, agg_sh, sid * RPT)

        @pl.when(sid < nfull)
        def _():
            pltpu.sync_copy(hs_hbm.at[pl.ds(sid * RPT, RPT)],
                            hs_sh.at[pl.ds(sid * RPT, RPT)])

        @pl.when(sid == nfull)
        def _():
            pltpu.sync_copy(hs_hbm.at[pl.ds(nfull * RPT, tail)],
                            hs_sh.at[pl.ds(nfull * RPT, tail)])

        plsc.subcore_barrier()

        def gather(j, b):
            pltpu.async_copy(hs_sh.at[src_v.at[j]], rows_v.at[b], gsems[b])

        for b in range(NBUF):
            gather(b, b)

        def grp(g, carry):
            for b in range(NBUF):
                j = g * NBUF + b
                pltpu.make_async_copy(hs0_hbm.at[pl.ds(0, K)], rows_v.at[b],
                                      gsems[b]).wait()
                pltpu.async_copy(rows_v.at[b], agg_sh.at[dst_v.at[j]],
                                 ssems[b], add=True)
            for b in range(NBUF):
                pltpu.make_async_copy(rows_v.at[b], agg_sh.at[pl.ds(0, K)],
                                      ssems[b]).wait()

                @pl.when(g + 1 < GRPS)
                def _():
                    gather((g + 1) * NBUF + b, b)

            return carry

        lax.fori_loop(0, GRPS, grp, 0)
        plsc.subcore_barrier()
        pltpu.sync_copy(
            agg_sh.at[pl.ds(sid * RPT, RPT)],
            out_hbm.at[phase, cid, pl.ds(sid * RPT, RPT)],
        )


_RB = 1000


def _deg_block(deg_ref):
    return deg_ref[0, :, 0:1] + deg_ref[1, :, 0:1] + 1.0


def _mm_body(x_ref, w_ref, h_ref):
    h_ref[...] = jnp.dot(x_ref[...], w_ref[...],
                         preferred_element_type=jnp.float32)


def _tc_matmul(x, w):
    grid = (N_NODES // _RB,)
    return pl.pallas_call(
        _mm_body,
        grid=grid,
        in_specs=[
            pl.BlockSpec((_RB, IN_CH), lambda i: (i, 0)),
            pl.BlockSpec((IN_CH, OUT_CH), lambda i: (0, 0)),
        ],
        out_specs=pl.BlockSpec((_RB, OUT_CH), lambda i: (i, 0)),
        out_shape=jax.ShapeDtypeStruct((N_NODES, OUT_CH), jnp.float32),
    )(x, w)


def _hs_body(h_ref, deg_ref, hs0_ref, hs1_ref):
    dis = lax.rsqrt(_deg_block(deg_ref))
    hs = h_ref[...] * dis
    hs0_ref[...] = hs[:, :HCH]
    hs1_ref[...] = hs[:, HCH:]


def _tc_hs(h, deg_parts):
    grid = (N_NODES // _RB,)
    half = jax.ShapeDtypeStruct((N_NODES, HCH), jnp.float32)
    return pl.pallas_call(
        _hs_body,
        grid=grid,
        in_specs=[
            pl.BlockSpec((_RB, OUT_CH), lambda i: (i, 0)),
            pl.BlockSpec((2, _RB, 16), lambda i: (0, i, 0)),
        ],
        out_specs=[
            pl.BlockSpec((_RB, HCH), lambda i: (i, 0)),
            pl.BlockSpec((_RB, HCH), lambda i: (i, 0)),
        ],
        out_shape=[half, half],
    )(h, deg_parts)


def _epi_body(agg_ref, hs0_ref, hs1_ref, deg_ref, b_ref, out_ref):
    dis = lax.rsqrt(_deg_block(deg_ref))
    agg = jnp.concatenate(
        [agg_ref[0, 0] + agg_ref[0, 1], agg_ref[1, 0] + agg_ref[1, 1]], axis=-1)
    hs = jnp.concatenate([hs0_ref[...], hs1_ref[...]], axis=-1)
    s = (agg + hs) * dis + b_ref[...]
    s = jnp.maximum(s, 0.0)
    m = jnp.max(s, axis=-1, keepdims=True)
    lse = jnp.log(jnp.sum(jnp.exp(s - m), axis=-1, keepdims=True)) + m
    out_ref[...] = s - lse


def _tc_epilogue(agg_parts, hs0, hs1, deg_parts, b):
    grid = (N_NODES // _RB,)
    return pl.pallas_call(
        _epi_body,
        grid=grid,
        in_specs=[
            pl.BlockSpec((2, 2, _RB, HCH), lambda i: (0, 0, i, 0)),
            pl.BlockSpec((_RB, HCH), lambda i: (i, 0)),
            pl.BlockSpec((_RB, HCH), lambda i: (i, 0)),
            pl.BlockSpec((2, _RB, 16), lambda i: (0, i, 0)),
            pl.BlockSpec((1, OUT_CH), lambda i: (0, 0)),
        ],
        out_specs=pl.BlockSpec((_RB, OUT_CH), lambda i: (i, 0)),
        out_shape=jax.ShapeDtypeStruct((N_NODES, OUT_CH), jnp.float32),
    )(agg_parts, hs0, hs1, deg_parts, b)


def kernel(x, edge_index, W, b):
    ei = edge_index.astype(jnp.int32)
    pad = E_PAD - N_EDGES
    src = jnp.concatenate([ei[0], jnp.zeros((pad,), jnp.int32)])
    dst = jnp.concatenate([ei[1], jnp.full((pad,), N_NODES, jnp.int32)])
    src = src.reshape(NW, BPW, K)
    dst = dst.reshape(NW, BPW, K)

    ones_rows = jnp.ones((K, 16), jnp.float32)
    zeros16 = jnp.zeros((K, 16), jnp.float32)
    zerosh = jnp.zeros((K, HCH), jnp.float32)

    h = _tc_matmul(x, W)
    deg_parts = _sc_degree(dst, ones_rows, zeros16)
    hs0, hs1 = _tc_hs(h, deg_parts)
    agg_parts = _sc_aggregate(src, dst, hs0, hs1, zerosh)
    return _tc_epilogue(agg_parts, hs0, hs1, deg_parts, b.reshape(1, OUT_CH))
